# Initial kernel scaffold; baseline (speedup 1.0000x reference)
import jax
import jax.numpy as jnp
from jax.experimental import pallas as pl


def kernel(ind_2, px):
    raise NotImplementedError("write your pallas kernel here")


# speedup vs baseline: 2.5141x; 2.5141x over previous
import functools

import jax
import jax.numpy as jnp
from jax import lax
from jax.experimental import pallas as pl
from jax.experimental.pallas import tpu as pltpu
from jax.experimental.pallas import tpu_sc as plsc

N_NODES = 10000
N_EDGES = 320000
D_FEAT = 128

NUM_CORES = 2
NUM_SUBCORES = 16
NW = NUM_CORES * NUM_SUBCORES
PER_W = N_EDGES // NW
CHUNK = 128
NFULL = PER_W // CHUNK
TAIL = PER_W - NFULL * CHUNK
NCHUNK = NFULL + 1
PER_W_PAD = NCHUNK * CHUNK


def _gather_kernel(idx_hbm, px_hbm, out_hbm, idx_v, rows_v, sem):
    wid = lax.axis_index("s") * NUM_CORES + lax.axis_index("c")
    base = wid * PER_W
    pltpu.sync_copy(idx_hbm.at[wid], idx_v)

    def body(i, _):
        pltpu.async_copy(px_hbm.at[idx_v.at[i]], rows_v, sem).wait()
        @pl.when(i < NFULL)
        def _full():
            pltpu.sync_copy(rows_v, out_hbm.at[pl.ds(base + i * CHUNK, CHUNK)])

        @pl.when(i == NFULL)
        def _tail():
            pltpu.sync_copy(
                rows_v.at[pl.ds(0, TAIL)],
                out_hbm.at[pl.ds(base + NFULL * CHUNK, TAIL)],
            )
        return 0

    lax.fori_loop(0, NCHUNK, body, 0)


@jax.jit
def _pix_gather(ind_j, px):
    mesh = plsc.VectorSubcoreMesh(core_axis_name="c", subcore_axis_name="s")
    run = functools.partial(
        pl.kernel,
        mesh=mesh,
        out_type=jax.ShapeDtypeStruct((N_EDGES, D_FEAT), jnp.float32),
        scratch_types=[
            pltpu.VMEM((NCHUNK, CHUNK), jnp.int32),
            pltpu.VMEM((CHUNK, D_FEAT), jnp.float32),
            pltpu.SemaphoreType.DMA,
        ],
    )(_gather_kernel)
    idx = ind_j.reshape(NW, PER_W)
    idx = jnp.pad(idx, ((0, 0), (0, PER_W_PAD - PER_W)))
    return run(idx.reshape(NW, NCHUNK, CHUNK), px)


def kernel(ind_2, px):
    return _pix_gather(ind_2[:, 1], px)
